# baseline (speedup 1.0000x reference)
import jax
import jax.numpy as jnp
from jax.experimental import pallas as pl
from jax.experimental.pallas import tpu as pltpu

_S = 16
_TB = 256
_G = 8
_TB2 = 2048


def _gcn_body(feat_ref, adj_ref, w_ref, bias_ref, a_ref, node_ref, ctx_ref):
    S = _S
    R = _G * S
    mapped = jnp.dot(feat_ref[...], w_ref[...],
                     preferred_element_type=jnp.float32)
    bias = bias_ref[...]
    a = a_ref[0]

    bi = jax.lax.broadcasted_iota(jnp.int32, (R, R), 0) // S
    bj = jax.lax.broadcasted_iota(jnp.int32, (R, R), 1) // S
    mask = (bi == bj).astype(jnp.float32)

    ri = jax.lax.broadcasted_iota(jnp.int32, (2 * _G, R), 0)
    cj = jax.lax.broadcasted_iota(jnp.int32, (2 * _G, R), 1)
    cb = cj // S
    cs = cj - cb * S
    is_ctx = ri < _G
    tgt = jnp.where(is_ctx, ri, ri - _G)
    pool = jnp.where((cb == tgt) & is_ctx & (cs < S - 1), 1.0 / (S - 1), 0.0)
    pool = pool + jnp.where((cb == tgt) & (~is_ctx) & (cs == S - 1), 1.0, 0.0)

    for g in range(_TB // _G):
        m_g = mapped[g * R:(g + 1) * R, :]
        a_g = adj_ref[g * R:(g + 1) * R, :]
        bd = jnp.concatenate([a_g] * _G, axis=1) * mask
        out = jnp.dot(bd, m_g, preferred_element_type=jnp.float32) + bias
        h = jnp.where(out >= 0, out, a * out)
        po = jnp.dot(pool, h, preferred_element_type=jnp.float32)
        ctx_ref[g * _G:(g + 1) * _G, :] = po[:_G]
        node_ref[g * _G:(g + 1) * _G, :] = po[_G:]


def _bil_body(node_ref, ctx_ref, ctxs_ref, w_ref, b_ref, pos_ref, neg_ref):
    t = jnp.dot(node_ref[...], w_ref[...], preferred_element_type=jnp.float32)
    b = b_ref[0]
    pos_ref[...] = jnp.sum(t * ctx_ref[...], axis=1, keepdims=True) + b
    neg_ref[...] = jnp.sum(t * ctxs_ref[...], axis=1, keepdims=True) + b


def kernel(feature_seq, adj_matrix, W_gcn, gcn_bias, prelu_a, bilinear_W,
           bilinear_b):
    B, S, IN = feature_seq.shape
    HID = W_gcn.shape[0]
    feat2 = feature_seq.reshape(B * S, IN)
    adj2 = adj_matrix.reshape(B * S, S)
    w_t = W_gcn.T
    bias2 = gcn_bias.reshape(1, HID)
    a1 = prelu_a.reshape(1)

    node_h, context_c = pl.pallas_call(
        _gcn_body,
        grid=(B // _TB,),
        in_specs=[
            pl.BlockSpec((_TB * S, IN), lambda i: (i, 0)),
            pl.BlockSpec((_TB * S, S), lambda i: (i, 0)),
            pl.BlockSpec((IN, HID), lambda i: (0, 0)),
            pl.BlockSpec((1, HID), lambda i: (0, 0)),
            pl.BlockSpec(memory_space=pltpu.SMEM),
        ],
        out_specs=[
            pl.BlockSpec((_TB, HID), lambda i: (i, 0)),
            pl.BlockSpec((_TB, HID), lambda i: (i, 0)),
        ],
        out_shape=[
            jax.ShapeDtypeStruct((B, HID), jnp.float32),
            jax.ShapeDtypeStruct((B, HID), jnp.float32),
        ],
    )(feat2, adj2, w_t, bias2, a1)

    ctx_shift = jnp.concatenate([context_c[B - 2:B - 1], context_c[:B - 1]],
                                axis=0)
    wb = bilinear_W.reshape(HID, HID)
    bb = bilinear_b.reshape(1)

    pos, neg = pl.pallas_call(
        _bil_body,
        grid=(B // _TB2,),
        in_specs=[
            pl.BlockSpec((_TB2, HID), lambda i: (i, 0)),
            pl.BlockSpec((_TB2, HID), lambda i: (i, 0)),
            pl.BlockSpec((_TB2, HID), lambda i: (i, 0)),
            pl.BlockSpec((HID, HID), lambda i: (0, 0)),
            pl.BlockSpec(memory_space=pltpu.SMEM),
        ],
        out_specs=[
            pl.BlockSpec((_TB2, 1), lambda i: (i, 0)),
            pl.BlockSpec((_TB2, 1), lambda i: (i, 0)),
        ],
        out_shape=[
            jax.ShapeDtypeStruct((B, 1), jnp.float32),
            jax.ShapeDtypeStruct((B, 1), jnp.float32),
        ],
    )(node_h, context_c, ctx_shift, wb, bb)

    logits = jnp.concatenate([pos, neg], axis=0)
    return (logits, node_h, context_c)


# speedup vs baseline: 1.0633x; 1.0633x over previous
import jax
import jax.numpy as jnp
from jax.experimental import pallas as pl
from jax.experimental.pallas import tpu as pltpu

_S = 16
_TB = 128
_TB2 = 2048


def _gcn_body(feat_ref, adjt_ref, w_ref, bias_ref, a_ref, node_ref, ctx_ref):
    S = _S
    TB = _TB
    w = w_ref[...]
    hid, in_dim = w.shape
    a = a_ref[0]
    bias = jnp.broadcast_to(bias_ref[...], (hid, TB))
    dn = (((1,), (1,)), ((), ()))

    m = []
    for t in range(S):
        ft = feat_ref[:, t * in_dim:(t + 1) * in_dim]
        m.append(jax.lax.dot_general(
            w, ft, dn, preferred_element_type=jnp.float32))

    ctx_acc = jnp.zeros((hid, TB), jnp.float32)
    for s in range(S):
        acc = bias
        for t in range(S):
            row = adjt_ref[s * S + t:s * S + t + 1, :]
            acc = acc + jnp.broadcast_to(row, (hid, TB)) * m[t]
        h = jnp.where(acc >= 0, acc, a * acc)
        if s < S - 1:
            ctx_acc = ctx_acc + h
        else:
            node_ref[...] = h
    ctx_ref[...] = ctx_acc * (1.0 / (S - 1))


def _bil_body(node_ref, ctx_ref, ctxs_ref, wt_ref, b_ref, pos_ref, neg_ref):
    t = jnp.dot(wt_ref[...], node_ref[...],
                preferred_element_type=jnp.float32)
    b = b_ref[0]
    pos_ref[...] = jnp.sum(t * ctx_ref[...], axis=0, keepdims=True) + b
    neg_ref[...] = jnp.sum(t * ctxs_ref[...], axis=0, keepdims=True) + b


def kernel(feature_seq, adj_matrix, W_gcn, gcn_bias, prelu_a, bilinear_W,
           bilinear_b):
    B, S, IN = feature_seq.shape
    HID = W_gcn.shape[0]
    feat2 = feature_seq.reshape(B, S * IN)
    adjt = adj_matrix.reshape(B, S * S).T
    bias2 = gcn_bias.reshape(HID, 1)
    a1 = prelu_a.reshape(1)

    node_t, ctx_t = pl.pallas_call(
        _gcn_body,
        grid=(B // _TB,),
        in_specs=[
            pl.BlockSpec((_TB, S * IN), lambda i: (i, 0)),
            pl.BlockSpec((S * S, _TB), lambda i: (0, i)),
            pl.BlockSpec((HID, IN), lambda i: (0, 0)),
            pl.BlockSpec((HID, 1), lambda i: (0, 0)),
            pl.BlockSpec(memory_space=pltpu.SMEM),
        ],
        out_specs=[
            pl.BlockSpec((HID, _TB), lambda i: (0, i)),
            pl.BlockSpec((HID, _TB), lambda i: (0, i)),
        ],
        out_shape=[
            jax.ShapeDtypeStruct((HID, B), jnp.float32),
            jax.ShapeDtypeStruct((HID, B), jnp.float32),
        ],
    )(feat2, adjt, W_gcn, bias2, a1)

    ctxs_t = jnp.concatenate([ctx_t[:, B - 2:B - 1], ctx_t[:, :B - 1]], axis=1)
    wbt = bilinear_W.reshape(HID, HID).T
    bb = bilinear_b.reshape(1)

    pos_t, neg_t = pl.pallas_call(
        _bil_body,
        grid=(B // _TB2,),
        in_specs=[
            pl.BlockSpec((HID, _TB2), lambda i: (0, i)),
            pl.BlockSpec((HID, _TB2), lambda i: (0, i)),
            pl.BlockSpec((HID, _TB2), lambda i: (0, i)),
            pl.BlockSpec((HID, HID), lambda i: (0, 0)),
            pl.BlockSpec(memory_space=pltpu.SMEM),
        ],
        out_specs=[
            pl.BlockSpec((1, _TB2), lambda i: (0, i)),
            pl.BlockSpec((1, _TB2), lambda i: (0, i)),
        ],
        out_shape=[
            jax.ShapeDtypeStruct((1, B), jnp.float32),
            jax.ShapeDtypeStruct((1, B), jnp.float32),
        ],
    )(node_t, ctx_t, ctxs_t, wbt, bb)

    logits = jnp.concatenate([pos_t, neg_t], axis=0).reshape(2 * B, 1)
    return (logits, node_t.T, ctx_t.T)
